# initial kernel scaffold (unmeasured)
import jax
import jax.numpy as jnp
from jax import lax
from jax.experimental import pallas as pl
from jax.experimental.pallas import tpu as pltpu

N_DEV = 4
M, N = 8192, 4096
HALF = M // 2
CHUNK = HALF // N_DEV
TILE = 512
DELTAS = (1, -1)


def kernel(x, w_mat):
    partial = jnp.dot(
        x.astype(jnp.bfloat16),
        w_mat.astype(jnp.bfloat16),
        preferred_element_type=jnp.float32,
    )

    def body(p_ref, out_ref, comm_ref, va, vb, vc, sem_a, sem_b, sem_c,
             send_sems, recv_sems):
        my = lax.axis_index("i")

        barrier = pltpu.get_barrier_semaphore()
        for delta in DELTAS:
            pl.semaphore_signal(
                barrier, inc=1,
                device_id=((my + delta) % N_DEV,),
                device_id_type=pl.DeviceIdType.MESH,
            )
        pl.semaphore_wait(barrier, 2)

        def row0(di, c):
            return di * HALF + c * CHUNK

        def add_chunk(mk_a, mk_b, mk_d):
            for r in range(0, CHUNK, TILE):
                ca = pltpu.make_async_copy(mk_a(r), va, sem_a)
                cb = pltpu.make_async_copy(mk_b(r), vb, sem_b)
                ca.start()
                cb.start()
                ca.wait()
                cb.wait()
                vc[...] = va[...] + vb[...]
                cc = pltpu.make_async_copy(vc, mk_d(r), sem_c)
                cc.start()
                cc.wait()

        for s in range(N_DEV - 1):
            rdmas = []
            for di, delta in enumerate(DELTAS):
                c_send = (my - delta * s) % N_DEV
                if s == 0:
                    src = p_ref.at[pl.ds(row0(di, c_send), CHUNK), :]
                else:
                    src = comm_ref.at[di, s - 1]
                rdma = pltpu.make_async_remote_copy(
                    src_ref=src,
                    dst_ref=comm_ref.at[di, s],
                    send_sem=send_sems.at[di, s],
                    recv_sem=recv_sems.at[di, s],
                    device_id=((my + delta) % N_DEV,),
                    device_id_type=pl.DeviceIdType.MESH,
                )
                rdma.start()
                rdmas.append(rdma)
            for di, delta in enumerate(DELTAS):
                rdmas[di].wait()
                c_recv = (my - delta * (s + 1)) % N_DEV
                if s < N_DEV - 2:
                    add_chunk(
                        lambda r, di=di, s=s: comm_ref.at[di, s, pl.ds(r, TILE), :],
                        lambda r, di=di, c=c_recv: p_ref.at[
                            pl.ds(row0(di, c) + r, TILE), :],
                        lambda r, di=di, s=s: comm_ref.at[di, s, pl.ds(r, TILE), :],
                    )
                else:
                    add_chunk(
                        lambda r, di=di, s=s: comm_ref.at[di, s, pl.ds(r, TILE), :],
                        lambda r, di=di, c=c_recv: p_ref.at[
                            pl.ds(row0(di, c) + r, TILE), :],
                        lambda r, di=di, c=c_recv: out_ref.at[
                            pl.ds(row0(di, c) + r, TILE), :],
                    )

        for t in range(N_DEV - 1):
            rdmas = []
            for di, delta in enumerate(DELTAS):
                c_send = (my + delta * (1 - t)) % N_DEV
                sl = pl.ds(row0(di, c_send), CHUNK)
                rdma = pltpu.make_async_remote_copy(
                    src_ref=out_ref.at[sl, :],
                    dst_ref=out_ref.at[sl, :],
                    send_sem=send_sems.at[di, N_DEV - 1 + t],
                    recv_sem=recv_sems.at[di, N_DEV - 1 + t],
                    device_id=((my + delta) % N_DEV,),
                    device_id_type=pl.DeviceIdType.MESH,
                )
                rdma.start()
                rdmas.append(rdma)
            for rdma in rdmas:
                rdma.wait()

    out, _comm = pl.pallas_call(
        body,
        out_shape=[
            jax.ShapeDtypeStruct((M, N), jnp.float32),
            jax.ShapeDtypeStruct((2, N_DEV - 1, CHUNK, N), jnp.float32),
        ],
        in_specs=[pl.BlockSpec(memory_space=pltpu.ANY)],
        out_specs=[
            pl.BlockSpec(memory_space=pltpu.ANY),
            pl.BlockSpec(memory_space=pltpu.ANY),
        ],
        scratch_shapes=[
            pltpu.VMEM((TILE, N), jnp.float32),
            pltpu.VMEM((TILE, N), jnp.float32),
            pltpu.VMEM((TILE, N), jnp.float32),
            pltpu.SemaphoreType.DMA,
            pltpu.SemaphoreType.DMA,
            pltpu.SemaphoreType.DMA,
            pltpu.SemaphoreType.DMA((2, 2 * (N_DEV - 1))),
            pltpu.SemaphoreType.DMA((2, 2 * (N_DEV - 1))),
        ],
        compiler_params=pltpu.CompilerParams(collective_id=0),
    )(partial)
    return out


# baseline (device time: 1503717 ns/iter reference)
import jax
import jax.numpy as jnp
from jax import lax
from jax.experimental import pallas as pl
from jax.experimental.pallas import tpu as pltpu

N_DEV = 4
M, N = 8192, 4096
HALF = M // 2
CHUNK = HALF // N_DEV
TILE = 512
DELTAS = (1, -1)


def kernel(x, w_mat):
    partial = jnp.dot(
        x.astype(jnp.bfloat16),
        w_mat.astype(jnp.bfloat16),
        preferred_element_type=jnp.float32,
    )

    def body(p_ref, out_ref, comm_ref, va, vb, vc, sem_a, sem_b, sem_c,
             send_sems, recv_sems):
        my = lax.axis_index("i")

        barrier = pltpu.get_barrier_semaphore()
        for delta in DELTAS:
            pl.semaphore_signal(
                barrier, inc=1,
                device_id=((my + delta) % N_DEV,),
                device_id_type=pl.DeviceIdType.MESH,
            )
        pl.semaphore_wait(barrier, 2)

        def row0(di, c):
            return di * HALF + c * CHUNK

        def add_chunk(mk_a, mk_b, mk_d):
            for r in range(0, CHUNK, TILE):
                ca = pltpu.make_async_copy(mk_a(r), va, sem_a)
                cb = pltpu.make_async_copy(mk_b(r), vb, sem_b)
                ca.start()
                cb.start()
                ca.wait()
                cb.wait()
                vc[...] = va[...] + vb[...]
                cc = pltpu.make_async_copy(vc, mk_d(r), sem_c)
                cc.start()
                cc.wait()

        for s in range(N_DEV - 1):
            rdmas = []
            for di, delta in enumerate(DELTAS):
                c_send = (my - delta * s) % N_DEV
                if s == 0:
                    src = p_ref.at[pl.ds(row0(di, c_send), CHUNK), :]
                else:
                    src = comm_ref.at[di, s - 1]
                rdma = pltpu.make_async_remote_copy(
                    src_ref=src,
                    dst_ref=comm_ref.at[di, s],
                    send_sem=send_sems.at[di, s],
                    recv_sem=recv_sems.at[di, s],
                    device_id=((my + delta) % N_DEV,),
                    device_id_type=pl.DeviceIdType.MESH,
                )
                rdma.start()
                rdmas.append(rdma)
            for di, delta in enumerate(DELTAS):
                rdmas[di].wait()
                c_recv = (my - delta * (s + 1)) % N_DEV
                if s < N_DEV - 2:
                    add_chunk(
                        lambda r, di=di, s=s: comm_ref.at[di, s, pl.ds(r, TILE), :],
                        lambda r, di=di, c=c_recv: p_ref.at[
                            pl.ds(row0(di, c) + r, TILE), :],
                        lambda r, di=di, s=s: comm_ref.at[di, s, pl.ds(r, TILE), :],
                    )
                else:
                    add_chunk(
                        lambda r, di=di, s=s: comm_ref.at[di, s, pl.ds(r, TILE), :],
                        lambda r, di=di, c=c_recv: p_ref.at[
                            pl.ds(row0(di, c) + r, TILE), :],
                        lambda r, di=di, c=c_recv: out_ref.at[
                            pl.ds(row0(di, c) + r, TILE), :],
                    )

        for t in range(N_DEV - 1):
            rdmas = []
            for di, delta in enumerate(DELTAS):
                c_send = (my + delta * (1 - t)) % N_DEV
                sl = pl.ds(row0(di, c_send), CHUNK)
                rdma = pltpu.make_async_remote_copy(
                    src_ref=out_ref.at[sl, :],
                    dst_ref=out_ref.at[sl, :],
                    send_sem=send_sems.at[di, N_DEV - 1 + t],
                    recv_sem=recv_sems.at[di, N_DEV - 1 + t],
                    device_id=((my + delta) % N_DEV,),
                    device_id_type=pl.DeviceIdType.MESH,
                )
                rdma.start()
                rdmas.append(rdma)
            for rdma in rdmas:
                rdma.wait()

    out, _comm = pl.pallas_call(
        body,
        out_shape=[
            jax.ShapeDtypeStruct((M, N), jnp.float32),
            jax.ShapeDtypeStruct((2, N_DEV - 1, CHUNK, N), jnp.float32),
        ],
        in_specs=[pl.BlockSpec(memory_space=pl.ANY)],
        out_specs=[
            pl.BlockSpec(memory_space=pl.ANY),
            pl.BlockSpec(memory_space=pl.ANY),
        ],
        scratch_shapes=[
            pltpu.VMEM((TILE, N), jnp.float32),
            pltpu.VMEM((TILE, N), jnp.float32),
            pltpu.VMEM((TILE, N), jnp.float32),
            pltpu.SemaphoreType.DMA,
            pltpu.SemaphoreType.DMA,
            pltpu.SemaphoreType.DMA,
            pltpu.SemaphoreType.DMA((2, 2 * (N_DEV - 1))),
            pltpu.SemaphoreType.DMA((2, 2 * (N_DEV - 1))),
        ],
        compiler_params=pltpu.CompilerParams(collective_id=0),
    )(partial)
    return out


# device time: 1425966 ns/iter; 1.0545x vs baseline; 1.0545x over previous
import jax
import jax.numpy as jnp
from jax import lax
from jax.experimental import pallas as pl
from jax.experimental.pallas import tpu as pltpu

N_DEV = 4
M, N = 8192, 4096
K_SH = 2048
HALF = M // 2
CHUNK = HALF // N_DEV
TILE = 256
DELTAS = (1, -1)


def kernel(x, w_mat):
    x16 = x.astype(jnp.bfloat16)
    w16 = w_mat.astype(jnp.bfloat16)

    def body(x_ref, w_ref, out_ref, p_ref, comm_ref, vw, vx, vo, va, vb, vc,
             sem_w, sem_x, sem_o, sem_a, sem_b, sem_c, send_sems, recv_sems):
        my = lax.axis_index("i")

        barrier = pltpu.get_barrier_semaphore()
        for delta in DELTAS:
            pl.semaphore_signal(
                barrier, inc=1,
                device_id=((my + delta) % N_DEV,),
                device_id_type=pl.DeviceIdType.MESH,
            )
        pl.semaphore_wait(barrier, 2)

        def row0(di, c):
            return di * HALF + (c % N_DEV) * CHUNK

        def add_chunk(mk_a, mk_b, mk_d):
            for r in range(0, CHUNK, TILE):
                ca = pltpu.make_async_copy(mk_a(r), va, sem_a)
                cb = pltpu.make_async_copy(mk_b(r), vb, sem_b)
                ca.start()
                cb.start()
                ca.wait()
                cb.wait()
                vc[...] = va[...] + vb[...]
                cc = pltpu.make_async_copy(vc, mk_d(r), sem_c)
                cc.start()
                cc.wait()

        def hop_rdmas(s):
            rdmas = []
            for di, delta in enumerate(DELTAS):
                c_send = (my - delta * s) % N_DEV
                if s == 0:
                    src = p_ref.at[pl.ds(row0(di, c_send), CHUNK), :]
                else:
                    src = comm_ref.at[di, s - 1]
                rdmas.append(pltpu.make_async_remote_copy(
                    src_ref=src,
                    dst_ref=comm_ref.at[di, s],
                    send_sem=send_sems.at[di, s],
                    recv_sem=recv_sems.at[di, s],
                    device_id=((my + delta) % N_DEV,),
                    device_id_type=pl.DeviceIdType.MESH,
                ))
            return rdmas

        def chunk_at(j, slot):
            if slot == 0:
                off = jnp.where(j < N_DEV - 1, -j, 1)
            else:
                off = jnp.where(j < N_DEV - 1, j, -1)
            return (my + off) % N_DEV

        def xcopy(j, slot):
            return pltpu.make_async_copy(
                x_ref.at[pl.ds(row0(slot, chunk_at(j, slot)), CHUNK), :],
                vx.at[slot],
                sem_x.at[slot],
            )

        cw = pltpu.make_async_copy(w_ref, vw, sem_w)
        cw.start()
        xcopy(0, 0).start()
        cw.wait()

        def gemm_iter(j, _):
            xcopy(j, 1).start()
            for slot in range(2):
                xcopy(j, slot).wait()
                vo[...] = jnp.dot(
                    vx[slot], vw[...], preferred_element_type=jnp.float32)
                co = pltpu.make_async_copy(
                    vo,
                    p_ref.at[pl.ds(row0(slot, chunk_at(j, slot)), CHUNK), :],
                    sem_o)
                co.start()
                co.wait()
                if slot == 0:
                    @pl.when(j < N_DEV - 1)
                    def _():
                        xcopy(j + 1, 0).start()

            @pl.when(j == 0)
            def _():
                for rdma in hop_rdmas(0):
                    rdma.start()
            return None

        lax.fori_loop(0, N_DEV, gemm_iter, None)

        for s in range(N_DEV - 1):
            rdmas = hop_rdmas(s)
            if s > 0:
                for rdma in rdmas:
                    rdma.start()
            for di, delta in enumerate(DELTAS):
                rdmas[di].wait()
                c_recv = (my - delta * (s + 1)) % N_DEV
                if s < N_DEV - 2:
                    add_chunk(
                        lambda r, di=di, s=s: comm_ref.at[di, s, pl.ds(r, TILE), :],
                        lambda r, di=di, c=c_recv: p_ref.at[
                            pl.ds(row0(di, c) + r, TILE), :],
                        lambda r, di=di, s=s: comm_ref.at[di, s, pl.ds(r, TILE), :],
                    )
                else:
                    add_chunk(
                        lambda r, di=di, s=s: comm_ref.at[di, s, pl.ds(r, TILE), :],
                        lambda r, di=di, c=c_recv: p_ref.at[
                            pl.ds(row0(di, c) + r, TILE), :],
                        lambda r, di=di, c=c_recv: out_ref.at[
                            pl.ds(row0(di, c) + r, TILE), :],
                    )

        for t in range(N_DEV - 1):
            rdmas = []
            for di, delta in enumerate(DELTAS):
                c_send = (my + delta * (1 - t)) % N_DEV
                sl = pl.ds(row0(di, c_send), CHUNK)
                rdma = pltpu.make_async_remote_copy(
                    src_ref=out_ref.at[sl, :],
                    dst_ref=out_ref.at[sl, :],
                    send_sem=send_sems.at[di, N_DEV - 1 + t],
                    recv_sem=recv_sems.at[di, N_DEV - 1 + t],
                    device_id=((my + delta) % N_DEV,),
                    device_id_type=pl.DeviceIdType.MESH,
                )
                rdma.start()
                rdmas.append(rdma)
            for rdma in rdmas:
                rdma.wait()

    out, _p, _comm = pl.pallas_call(
        body,
        out_shape=[
            jax.ShapeDtypeStruct((M, N), jnp.float32),
            jax.ShapeDtypeStruct((M, N), jnp.float32),
            jax.ShapeDtypeStruct((2, N_DEV - 1, CHUNK, N), jnp.float32),
        ],
        in_specs=[
            pl.BlockSpec(memory_space=pl.ANY),
            pl.BlockSpec(memory_space=pl.ANY),
        ],
        out_specs=[
            pl.BlockSpec(memory_space=pl.ANY),
            pl.BlockSpec(memory_space=pl.ANY),
            pl.BlockSpec(memory_space=pl.ANY),
        ],
        scratch_shapes=[
            pltpu.VMEM((K_SH, N), jnp.bfloat16),
            pltpu.VMEM((2, CHUNK, K_SH), jnp.bfloat16),
            pltpu.VMEM((CHUNK, N), jnp.float32),
            pltpu.VMEM((TILE, N), jnp.float32),
            pltpu.VMEM((TILE, N), jnp.float32),
            pltpu.VMEM((TILE, N), jnp.float32),
            pltpu.SemaphoreType.DMA,
            pltpu.SemaphoreType.DMA((2,)),
            pltpu.SemaphoreType.DMA,
            pltpu.SemaphoreType.DMA,
            pltpu.SemaphoreType.DMA,
            pltpu.SemaphoreType.DMA,
            pltpu.SemaphoreType.DMA((2, 2 * (N_DEV - 1))),
            pltpu.SemaphoreType.DMA((2, 2 * (N_DEV - 1))),
        ],
        compiler_params=pltpu.CompilerParams(
            collective_id=0, vmem_limit_bytes=64 * 1024 * 1024),
    )(x16, w16)
    return out


# device time: 871242 ns/iter; 1.7259x vs baseline; 1.6367x over previous
import jax
import jax.numpy as jnp
from jax import lax
from jax.experimental import pallas as pl
from jax.experimental.pallas import tpu as pltpu

N_DEV = 4
M, N = 8192, 4096
K_SH = 2048
HALF = M // 2
CHUNK = HALF // N_DEV
TILE = 512
DELTAS = (1, -1)


def kernel(x, w_mat):
    x16 = x.astype(jnp.bfloat16)
    w16 = w_mat.astype(jnp.bfloat16)

    def body(x_ref, w_ref, out_ref, p_ref, g_ref, comm_ref,
             vw, vx, vo, va, vb, vc32, vc16,
             sem_w, sem_x, sem_o, sem_a, sem_b, sem_c, sem_d,
             send_sems, recv_sems):
        my = lax.axis_index("i")

        barrier = pltpu.get_barrier_semaphore()
        for delta in DELTAS:
            pl.semaphore_signal(
                barrier, inc=1,
                device_id=((my + delta) % N_DEV,),
                device_id_type=pl.DeviceIdType.MESH,
            )
        pl.semaphore_wait(barrier, 2)

        def row0(di, c):
            return di * HALF + (c % N_DEV) * CHUNK

        def hop_rdmas(s):
            rdmas = []
            for di, delta in enumerate(DELTAS):
                c_send = (my - delta * s) % N_DEV
                if s == 0:
                    src = p_ref.at[pl.ds(row0(di, c_send), CHUNK), :]
                else:
                    src = comm_ref.at[di, s - 1]
                rdmas.append(pltpu.make_async_remote_copy(
                    src_ref=src,
                    dst_ref=comm_ref.at[di, s],
                    send_sem=send_sems.at[di, s],
                    recv_sem=recv_sems.at[di, s],
                    device_id=((my + delta) % N_DEV,),
                    device_id_type=pl.DeviceIdType.MESH,
                ))
            return rdmas

        def ag_rdma(t, di):
            delta = DELTAS[di]
            sl = pl.ds(row0(di, (my + delta * (1 - t)) % N_DEV), CHUNK)
            return pltpu.make_async_remote_copy(
                src_ref=g_ref.at[sl, :],
                dst_ref=g_ref.at[sl, :],
                send_sem=send_sems.at[di, N_DEV - 1 + t],
                recv_sem=recv_sems.at[di, N_DEV - 1 + t],
                device_id=((my + delta) % N_DEV,),
                device_id_type=pl.DeviceIdType.MESH,
            )

        def chunk_at(j, slot):
            if slot == 0:
                off = jnp.where(j < N_DEV - 1, -j, 1)
            else:
                off = jnp.where(j < N_DEV - 1, j, -1)
            return (my + off) % N_DEV

        def xcopy(j, slot):
            return pltpu.make_async_copy(
                x_ref.at[pl.ds(row0(slot, chunk_at(j, slot)), CHUNK), :],
                vx.at[slot],
                sem_x.at[slot],
            )

        cw = pltpu.make_async_copy(w_ref, vw, sem_w)
        cw.start()
        xcopy(0, 0).start()
        cw.wait()

        def gemm_iter(j, _):
            xcopy(j, 1).start()
            for slot in range(2):
                xcopy(j, slot).wait()
                vo[...] = jnp.dot(
                    vx[slot], vw[...],
                    preferred_element_type=jnp.float32,
                ).astype(jnp.bfloat16)
                co = pltpu.make_async_copy(
                    vo,
                    p_ref.at[pl.ds(row0(slot, chunk_at(j, slot)), CHUNK), :],
                    sem_o)
                co.start()
                co.wait()
                if slot == 0:
                    @pl.when(j < N_DEV - 1)
                    def _():
                        xcopy(j + 1, 0).start()

            @pl.when(j == 0)
            def _():
                for rdma in hop_rdmas(0):
                    rdma.start()
            return None

        lax.fori_loop(0, N_DEV, gemm_iter, None)

        def add_chunk(mk_a, mk_b, mk_d16, mk_d32=None):
            for r in range(0, CHUNK, TILE):
                ca = pltpu.make_async_copy(mk_a(r), va, sem_a)
                cb = pltpu.make_async_copy(mk_b(r), vb, sem_b)
                ca.start()
                cb.start()
                ca.wait()
                cb.wait()
                vc32[...] = (va[...].astype(jnp.float32)
                             + vb[...].astype(jnp.float32))
                vc16[...] = vc32[...].astype(jnp.bfloat16)
                cc = pltpu.make_async_copy(vc16, mk_d16(r), sem_c)
                cc.start()
                if mk_d32 is not None:
                    cd = pltpu.make_async_copy(vc32, mk_d32(r), sem_d)
                    cd.start()
                    cd.wait()
                cc.wait()

        for s in range(N_DEV - 1):
            rdmas = hop_rdmas(s)
            nxt = hop_rdmas(s + 1) if s < N_DEV - 2 else None
            for di, delta in enumerate(DELTAS):
                rdmas[di].wait()
                c_recv = (my - delta * (s + 1)) % N_DEV
                if s < N_DEV - 2:
                    add_chunk(
                        lambda r, di=di, s=s: comm_ref.at[di, s, pl.ds(r, TILE), :],
                        lambda r, di=di, c=c_recv: p_ref.at[
                            pl.ds(row0(di, c) + r, TILE), :],
                        lambda r, di=di, s=s: comm_ref.at[di, s, pl.ds(r, TILE), :],
                    )
                    nxt[di].start()
                else:
                    add_chunk(
                        lambda r, di=di, s=s: comm_ref.at[di, s, pl.ds(r, TILE), :],
                        lambda r, di=di, c=c_recv: p_ref.at[
                            pl.ds(row0(di, c) + r, TILE), :],
                        lambda r, di=di, c=c_recv: g_ref.at[
                            pl.ds(row0(di, c) + r, TILE), :],
                        lambda r, di=di, c=c_recv: out_ref.at[
                            pl.ds(row0(di, c) + r, TILE), :],
                    )
                    ag_rdma(0, di).start()

        def cast_chunk(c_row0):
            for r in range(0, CHUNK, TILE):
                ca = pltpu.make_async_copy(
                    g_ref.at[pl.ds(c_row0 + r, TILE), :], va, sem_a)
                ca.start()
                ca.wait()
                vc32[...] = va[...].astype(jnp.float32)
                cd = pltpu.make_async_copy(
                    vc32, out_ref.at[pl.ds(c_row0 + r, TILE), :], sem_d)
                cd.start()
                cd.wait()

        for t in range(N_DEV - 1):
            for di in range(2):
                ag_rdma(t, di).wait()
                if t < N_DEV - 2:
                    ag_rdma(t + 1, di).start()
            for di, delta in enumerate(DELTAS):
                cast_chunk(row0(di, (my - delta * t) % N_DEV))

    out, _p, _g, _comm = pl.pallas_call(
        body,
        out_shape=[
            jax.ShapeDtypeStruct((M, N), jnp.float32),
            jax.ShapeDtypeStruct((M, N), jnp.bfloat16),
            jax.ShapeDtypeStruct((M, N), jnp.bfloat16),
            jax.ShapeDtypeStruct((2, N_DEV - 1, CHUNK, N), jnp.bfloat16),
        ],
        in_specs=[
            pl.BlockSpec(memory_space=pl.ANY),
            pl.BlockSpec(memory_space=pl.ANY),
        ],
        out_specs=[
            pl.BlockSpec(memory_space=pl.ANY),
            pl.BlockSpec(memory_space=pl.ANY),
            pl.BlockSpec(memory_space=pl.ANY),
            pl.BlockSpec(memory_space=pl.ANY),
        ],
        scratch_shapes=[
            pltpu.VMEM((K_SH, N), jnp.bfloat16),
            pltpu.VMEM((2, CHUNK, K_SH), jnp.bfloat16),
            pltpu.VMEM((CHUNK, N), jnp.bfloat16),
            pltpu.VMEM((TILE, N), jnp.bfloat16),
            pltpu.VMEM((TILE, N), jnp.bfloat16),
            pltpu.VMEM((TILE, N), jnp.float32),
            pltpu.VMEM((TILE, N), jnp.bfloat16),
            pltpu.SemaphoreType.DMA,
            pltpu.SemaphoreType.DMA((2,)),
            pltpu.SemaphoreType.DMA,
            pltpu.SemaphoreType.DMA,
            pltpu.SemaphoreType.DMA,
            pltpu.SemaphoreType.DMA,
            pltpu.SemaphoreType.DMA,
            pltpu.SemaphoreType.DMA((2, 2 * (N_DEV - 1))),
            pltpu.SemaphoreType.DMA((2, 2 * (N_DEV - 1))),
        ],
        compiler_params=pltpu.CompilerParams(
            collective_id=0, vmem_limit_bytes=64 * 1024 * 1024),
    )(x16, w16)
    return out


# device time: 799935 ns/iter; 1.8798x vs baseline; 1.0891x over previous
import jax
import jax.numpy as jnp
from jax import lax
from jax.experimental import pallas as pl
from jax.experimental.pallas import tpu as pltpu

N_DEV = 4
M, N = 8192, 4096
K_SH = 2048
HALF = M // 2
CHUNK = HALF // N_DEV
TILE = 512
DELTAS = (1, -1)


def kernel(x, w_mat):
    w16 = w_mat.astype(jnp.bfloat16)

    def body(x_ref, w_ref, out_ref, p_ref, g_ref, comm_ref,
             vw, vx, vo, va, vb, vc32, vc16,
             sem_w, sem_x, sem_o, sem_a, sem_b, sem_c, sem_d,
             send_sems, recv_sems):
        my = lax.axis_index("i")

        barrier = pltpu.get_barrier_semaphore()
        for delta in DELTAS:
            pl.semaphore_signal(
                barrier, inc=1,
                device_id=((my + delta) % N_DEV,),
                device_id_type=pl.DeviceIdType.MESH,
            )
        pl.semaphore_wait(barrier, 2)

        def row0(di, c):
            return di * HALF + (c % N_DEV) * CHUNK

        def hop_rdmas(s):
            rdmas = []
            for di, delta in enumerate(DELTAS):
                c_send = (my - delta * s) % N_DEV
                if s == 0:
                    src = p_ref.at[pl.ds(row0(di, c_send), CHUNK), :]
                else:
                    src = comm_ref.at[di, s - 1]
                rdmas.append(pltpu.make_async_remote_copy(
                    src_ref=src,
                    dst_ref=comm_ref.at[di, s],
                    send_sem=send_sems.at[di, s],
                    recv_sem=recv_sems.at[di, s],
                    device_id=((my + delta) % N_DEV,),
                    device_id_type=pl.DeviceIdType.MESH,
                ))
            return rdmas

        def ag_rdma(t, di):
            t = jnp.minimum(t, N_DEV - 2)
            delta = DELTAS[di]
            sl = pl.ds(row0(di, (my + delta * (1 - t)) % N_DEV), CHUNK)
            return pltpu.make_async_remote_copy(
                src_ref=g_ref.at[sl, :],
                dst_ref=g_ref.at[sl, :],
                send_sem=send_sems.at[di, N_DEV - 1 + t],
                recv_sem=recv_sems.at[di, N_DEV - 1 + t],
                device_id=((my + delta) % N_DEV,),
                device_id_type=pl.DeviceIdType.MESH,
            )

        def chunk_at(j, slot):
            if slot == 0:
                off = jnp.where(j < N_DEV - 1, -j, 1)
            else:
                off = jnp.where(j < N_DEV - 1, j, -1)
            return (my + off) % N_DEV

        def xcopy(j, slot):
            return pltpu.make_async_copy(
                x_ref.at[pl.ds(row0(slot, chunk_at(j, slot)), CHUNK), :],
                vx.at[slot],
                sem_x.at[slot],
            )

        cw = pltpu.make_async_copy(w_ref, vw, sem_w)
        cw.start()
        xcopy(0, 0).start()
        cw.wait()

        def add_chunk(mk_a, mk_b, mk_d16, mk_d32=None):
            for r in range(0, CHUNK, TILE):
                ca = pltpu.make_async_copy(mk_a(r), va, sem_a)
                cb = pltpu.make_async_copy(mk_b(r), vb, sem_b)
                ca.start()
                cb.start()
                ca.wait()
                cb.wait()
                vc32[...] = (va[...].astype(jnp.float32)
                             + vb[...].astype(jnp.float32))
                vc16[...] = vc32[...].astype(jnp.bfloat16)
                cc = pltpu.make_async_copy(vc16, mk_d16(r), sem_c)
                cc.start()
                if mk_d32 is not None:
                    cd = pltpu.make_async_copy(vc32, mk_d32(r), sem_d)
                    cd.start()
                    cd.wait()
                cc.wait()

        def gemm_iter(j, _):
            xcopy(j, 1).start()
            for slot in range(2):
                xcopy(j, slot).wait()
                vo[...] = jnp.dot(
                    vx[slot].astype(jnp.bfloat16), vw[...],
                    preferred_element_type=jnp.float32,
                ).astype(jnp.bfloat16)
                co = pltpu.make_async_copy(
                    vo,
                    p_ref.at[pl.ds(row0(slot, chunk_at(j, slot)), CHUNK), :],
                    sem_o)
                co.start()
                co.wait()
                if slot == 0:
                    @pl.when(j < N_DEV - 1)
                    def _():
                        xcopy(j + 1, 0).start()

            @pl.when(j == 0)
            def _():
                for rdma in hop_rdmas(0):
                    rdma.start()

            @pl.when(j == N_DEV - 2)
            def _():
                rd0 = hop_rdmas(0)
                rd1 = hop_rdmas(1)
                for di, delta in enumerate(DELTAS):
                    rd0[di].wait()
                    c_recv = (my - delta) % N_DEV
                    add_chunk(
                        lambda r, di=di: comm_ref.at[di, 0, pl.ds(r, TILE), :],
                        lambda r, di=di, c=c_recv: p_ref.at[
                            pl.ds(row0(di, c) + r, TILE), :],
                        lambda r, di=di: comm_ref.at[di, 0, pl.ds(r, TILE), :],
                    )
                    rd1[di].start()
            return None

        lax.fori_loop(0, N_DEV, gemm_iter, None)

        for s in range(1, N_DEV - 1):
            rdmas = hop_rdmas(s)
            nxt = hop_rdmas(s + 1) if s < N_DEV - 2 else None
            for di, delta in enumerate(DELTAS):
                rdmas[di].wait()
                c_recv = (my - delta * (s + 1)) % N_DEV
                if s < N_DEV - 2:
                    add_chunk(
                        lambda r, di=di, s=s: comm_ref.at[di, s, pl.ds(r, TILE), :],
                        lambda r, di=di, c=c_recv: p_ref.at[
                            pl.ds(row0(di, c) + r, TILE), :],
                        lambda r, di=di, s=s: comm_ref.at[di, s, pl.ds(r, TILE), :],
                    )
                    nxt[di].start()
                else:
                    add_chunk(
                        lambda r, di=di, s=s: comm_ref.at[di, s, pl.ds(r, TILE), :],
                        lambda r, di=di, c=c_recv: p_ref.at[
                            pl.ds(row0(di, c) + r, TILE), :],
                        lambda r, di=di, c=c_recv: g_ref.at[
                            pl.ds(row0(di, c) + r, TILE), :],
                        lambda r, di=di, c=c_recv: out_ref.at[
                            pl.ds(row0(di, c) + r, TILE), :],
                    )
                    ag_rdma(0, di).start()

        def cast_chunk(c_row0):
            for r in range(0, CHUNK, TILE):
                ca = pltpu.make_async_copy(
                    g_ref.at[pl.ds(c_row0 + r, TILE), :], va, sem_a)
                ca.start()
                ca.wait()
                vc32[...] = va[...].astype(jnp.float32)
                cd = pltpu.make_async_copy(
                    vc32, out_ref.at[pl.ds(c_row0 + r, TILE), :], sem_d)
                cd.start()
                cd.wait()

        def ag_iter(t, _):
            for di in range(2):
                ag_rdma(t, di).wait()

                @pl.when(t < N_DEV - 2)
                def _(di=di):
                    ag_rdma(t + 1, di).start()
            for di, delta in enumerate(DELTAS):
                cast_chunk(row0(di, (my - delta * t) % N_DEV))
            return None

        lax.fori_loop(0, N_DEV - 1, ag_iter, None)

    out, _p, _g, _comm = pl.pallas_call(
        body,
        out_shape=[
            jax.ShapeDtypeStruct((M, N), jnp.float32),
            jax.ShapeDtypeStruct((M, N), jnp.bfloat16),
            jax.ShapeDtypeStruct((M, N), jnp.bfloat16),
            jax.ShapeDtypeStruct((2, N_DEV - 1, CHUNK, N), jnp.bfloat16),
        ],
        in_specs=[
            pl.BlockSpec(memory_space=pl.ANY),
            pl.BlockSpec(memory_space=pl.ANY),
        ],
        out_specs=[
            pl.BlockSpec(memory_space=pl.ANY),
            pl.BlockSpec(memory_space=pl.ANY),
            pl.BlockSpec(memory_space=pl.ANY),
            pl.BlockSpec(memory_space=pl.ANY),
        ],
        scratch_shapes=[
            pltpu.VMEM((K_SH, N), jnp.bfloat16),
            pltpu.VMEM((2, CHUNK, K_SH), jnp.float32),
            pltpu.VMEM((CHUNK, N), jnp.bfloat16),
            pltpu.VMEM((TILE, N), jnp.bfloat16),
            pltpu.VMEM((TILE, N), jnp.bfloat16),
            pltpu.VMEM((TILE, N), jnp.float32),
            pltpu.VMEM((TILE, N), jnp.bfloat16),
            pltpu.SemaphoreType.DMA,
            pltpu.SemaphoreType.DMA((2,)),
            pltpu.SemaphoreType.DMA,
            pltpu.SemaphoreType.DMA,
            pltpu.SemaphoreType.DMA,
            pltpu.SemaphoreType.DMA,
            pltpu.SemaphoreType.DMA,
            pltpu.SemaphoreType.DMA((2, 2 * (N_DEV - 1))),
            pltpu.SemaphoreType.DMA((2, 2 * (N_DEV - 1))),
        ],
        compiler_params=pltpu.CompilerParams(
            collective_id=0, vmem_limit_bytes=64 * 1024 * 1024),
    )(x, w16)
    return out


# device time: 779112 ns/iter; 1.9300x vs baseline; 1.0267x over previous
import jax
import jax.numpy as jnp
from jax import lax
from jax.experimental import pallas as pl
from jax.experimental.pallas import tpu as pltpu

N_DEV = 4
M, N = 8192, 4096
K_SH = 2048
HALF = M // 2
CHUNK = HALF // N_DEV
TILE = 256
DELTAS = (1, -1)


def kernel(x, w_mat):
    w16 = w_mat.astype(jnp.bfloat16)

    def body(x_ref, w_ref, out_ref, p_ref, g_ref, comm_ref,
             vw, vx, vo, va, vb, vc32, vc16,
             sem_w, sem_x, sem_o, sem_a, sem_b, sem_c, sem_d,
             send_sems, recv_sems):
        my = lax.axis_index("i")

        barrier = pltpu.get_barrier_semaphore()
        for delta in DELTAS:
            pl.semaphore_signal(
                barrier, inc=1,
                device_id=((my + delta) % N_DEV,),
                device_id_type=pl.DeviceIdType.MESH,
            )
        pl.semaphore_wait(barrier, 2)

        def row0(di, c):
            return di * HALF + (c % N_DEV) * CHUNK

        def hop_rdmas(s):
            rdmas = []
            for di, delta in enumerate(DELTAS):
                c_send = (my - delta * s) % N_DEV
                if s == 0:
                    src = p_ref.at[pl.ds(row0(di, c_send), CHUNK), :]
                else:
                    src = comm_ref.at[di, s - 1]
                rdmas.append(pltpu.make_async_remote_copy(
                    src_ref=src,
                    dst_ref=comm_ref.at[di, s],
                    send_sem=send_sems.at[di, s],
                    recv_sem=recv_sems.at[di, s],
                    device_id=((my + delta) % N_DEV,),
                    device_id_type=pl.DeviceIdType.MESH,
                ))
            return rdmas

        def ag_rdma(t, di):
            t = jnp.minimum(t, N_DEV - 2)
            delta = DELTAS[di]
            sl = pl.ds(row0(di, (my + delta * (1 - t)) % N_DEV), CHUNK)
            return pltpu.make_async_remote_copy(
                src_ref=g_ref.at[sl, :],
                dst_ref=g_ref.at[sl, :],
                send_sem=send_sems.at[di, N_DEV - 1 + t],
                recv_sem=recv_sems.at[di, N_DEV - 1 + t],
                device_id=((my + delta) % N_DEV,),
                device_id_type=pl.DeviceIdType.MESH,
            )

        def chunk_at(j, slot):
            if slot == 0:
                off = jnp.where(j < N_DEV - 1, -j, 1)
            else:
                off = jnp.where(j < N_DEV - 1, j, -1)
            return (my + off) % N_DEV

        def xcopy(j, slot):
            return pltpu.make_async_copy(
                x_ref.at[pl.ds(row0(slot, chunk_at(j, slot)), CHUNK), :],
                vx.at[slot],
                sem_x.at[slot],
            )

        cw = pltpu.make_async_copy(w_ref, vw, sem_w)
        cw.start()
        xcopy(0, 0).start()
        cw.wait()

        NT = CHUNK // TILE

        def add_chunk(mk_a, mk_b, mk_d16, mk_d32=None):
            def ins(i):
                b = i % 2
                return (
                    pltpu.make_async_copy(mk_a(i * TILE), va.at[b], sem_a.at[b]),
                    pltpu.make_async_copy(mk_b(i * TILE), vb.at[b], sem_b.at[b]),
                )
            prev = [None, None]
            for c in ins(0):
                c.start()
            for i in range(NT):
                b = i % 2
                if i + 1 < NT:
                    for c in ins(i + 1):
                        c.start()
                for c in ins(i):
                    c.wait()
                if prev[b] is not None:
                    for c in prev[b]:
                        c.wait()
                vc32[b] = (va[b].astype(jnp.float32)
                           + vb[b].astype(jnp.float32))
                vc16[b] = vc32[b].astype(jnp.bfloat16)
                outs = [pltpu.make_async_copy(
                    vc16.at[b], mk_d16(i * TILE), sem_c.at[b])]
                if mk_d32 is not None:
                    outs.append(pltpu.make_async_copy(
                        vc32.at[b], mk_d32(i * TILE), sem_d.at[b]))
                for c in outs:
                    c.start()
                prev[b] = outs
            for pv in prev:
                if pv is not None:
                    for c in pv:
                        c.wait()

        def gemm_iter(j, _):
            xcopy(j, 1).start()
            for slot in range(2):
                xcopy(j, slot).wait()
                vo[...] = jnp.dot(
                    vx[slot].astype(jnp.bfloat16), vw[...],
                    preferred_element_type=jnp.float32,
                ).astype(jnp.bfloat16)
                co = pltpu.make_async_copy(
                    vo,
                    p_ref.at[pl.ds(row0(slot, chunk_at(j, slot)), CHUNK), :],
                    sem_o)
                co.start()
                co.wait()
                if slot == 0:
                    @pl.when(j < N_DEV - 1)
                    def _():
                        xcopy(j + 1, 0).start()

            @pl.when(j == 0)
            def _():
                for rdma in hop_rdmas(0):
                    rdma.start()

            @pl.when(j == N_DEV - 2)
            def _():
                rd0 = hop_rdmas(0)
                rd1 = hop_rdmas(1)
                for di, delta in enumerate(DELTAS):
                    rd0[di].wait()
                    c_recv = (my - delta) % N_DEV
                    add_chunk(
                        lambda r, di=di: comm_ref.at[di, 0, pl.ds(r, TILE), :],
                        lambda r, di=di, c=c_recv: p_ref.at[
                            pl.ds(row0(di, c) + r, TILE), :],
                        lambda r, di=di: comm_ref.at[di, 0, pl.ds(r, TILE), :],
                    )
                    rd1[di].start()
            return None

        lax.fori_loop(0, N_DEV, gemm_iter, None)

        for s in range(1, N_DEV - 1):
            rdmas = hop_rdmas(s)
            nxt = hop_rdmas(s + 1) if s < N_DEV - 2 else None
            for di, delta in enumerate(DELTAS):
                rdmas[di].wait()
                c_recv = (my - delta * (s + 1)) % N_DEV
                if s < N_DEV - 2:
                    add_chunk(
                        lambda r, di=di, s=s: comm_ref.at[di, s, pl.ds(r, TILE), :],
                        lambda r, di=di, c=c_recv: p_ref.at[
                            pl.ds(row0(di, c) + r, TILE), :],
                        lambda r, di=di, s=s: comm_ref.at[di, s, pl.ds(r, TILE), :],
                    )
                    nxt[di].start()
                else:
                    add_chunk(
                        lambda r, di=di, s=s: comm_ref.at[di, s, pl.ds(r, TILE), :],
                        lambda r, di=di, c=c_recv: p_ref.at[
                            pl.ds(row0(di, c) + r, TILE), :],
                        lambda r, di=di, c=c_recv: g_ref.at[
                            pl.ds(row0(di, c) + r, TILE), :],
                        lambda r, di=di, c=c_recv: out_ref.at[
                            pl.ds(row0(di, c) + r, TILE), :],
                    )
                    ag_rdma(0, di).start()

        def cast_chunk(c_row0):
            def ins(i):
                b = i % 2
                return pltpu.make_async_copy(
                    g_ref.at[pl.ds(c_row0 + i * TILE, TILE), :],
                    va.at[b], sem_a.at[b])
            prev = [None, None]
            ins(0).start()
            for i in range(NT):
                b = i % 2
                if i + 1 < NT:
                    ins(i + 1).start()
                ins(i).wait()
                if prev[b] is not None:
                    prev[b].wait()
                vc32[b] = va[b].astype(jnp.float32)
                cd = pltpu.make_async_copy(
                    vc32.at[b],
                    out_ref.at[pl.ds(c_row0 + i * TILE, TILE), :],
                    sem_d.at[b])
                cd.start()
                prev[b] = cd
            for pv in prev:
                if pv is not None:
                    pv.wait()

        def ag_iter(t, _):
            for di in range(2):
                ag_rdma(t, di).wait()

                @pl.when(t < N_DEV - 2)
                def _(di=di):
                    ag_rdma(t + 1, di).start()
            for di, delta in enumerate(DELTAS):
                cast_chunk(row0(di, (my - delta * t) % N_DEV))
            return None

        lax.fori_loop(0, N_DEV - 1, ag_iter, None)

    out, _p, _g, _comm = pl.pallas_call(
        body,
        out_shape=[
            jax.ShapeDtypeStruct((M, N), jnp.float32),
            jax.ShapeDtypeStruct((M, N), jnp.bfloat16),
            jax.ShapeDtypeStruct((M, N), jnp.bfloat16),
            jax.ShapeDtypeStruct((2, N_DEV - 1, CHUNK, N), jnp.bfloat16),
        ],
        in_specs=[
            pl.BlockSpec(memory_space=pl.ANY),
            pl.BlockSpec(memory_space=pl.ANY),
        ],
        out_specs=[
            pl.BlockSpec(memory_space=pl.ANY),
            pl.BlockSpec(memory_space=pl.ANY),
            pl.BlockSpec(memory_space=pl.ANY),
            pl.BlockSpec(memory_space=pl.ANY),
        ],
        scratch_shapes=[
            pltpu.VMEM((K_SH, N), jnp.bfloat16),
            pltpu.VMEM((2, CHUNK, K_SH), jnp.float32),
            pltpu.VMEM((CHUNK, N), jnp.bfloat16),
            pltpu.VMEM((2, TILE, N), jnp.bfloat16),
            pltpu.VMEM((2, TILE, N), jnp.bfloat16),
            pltpu.VMEM((2, TILE, N), jnp.float32),
            pltpu.VMEM((2, TILE, N), jnp.bfloat16),
            pltpu.SemaphoreType.DMA,
            pltpu.SemaphoreType.DMA((2,)),
            pltpu.SemaphoreType.DMA,
            pltpu.SemaphoreType.DMA((2,)),
            pltpu.SemaphoreType.DMA((2,)),
            pltpu.SemaphoreType.DMA((2,)),
            pltpu.SemaphoreType.DMA((2,)),
            pltpu.SemaphoreType.DMA((2, 2 * (N_DEV - 1))),
            pltpu.SemaphoreType.DMA((2, 2 * (N_DEV - 1))),
        ],
        compiler_params=pltpu.CompilerParams(
            collective_id=0, vmem_limit_bytes=64 * 1024 * 1024),
    )(x, w16)
    return out


# device time: 778390 ns/iter; 1.9318x vs baseline; 1.0009x over previous
import jax
import jax.numpy as jnp
from jax import lax
from jax.experimental import pallas as pl
from jax.experimental.pallas import tpu as pltpu

try:
    jax.config.update("jax_compilation_cache_dir", "/tmp/jax_persistent_cache")
    jax.config.update("jax_persistent_cache_min_compile_time_secs", 0)
    jax.config.update("jax_persistent_cache_min_entry_size_bytes", 0)
except Exception:
    pass

N_DEV = 4
M, N = 8192, 4096
K_SH = 2048
HALF = M // 2
CHUNK = HALF // N_DEV
TILE = 256
DELTAS = (1, -1)


def kernel(x, w_mat):
    w16 = w_mat.astype(jnp.bfloat16)

    def body(x_ref, w_ref, out_ref, p_ref, g_ref, comm_ref,
             vw, vx, vo, va, vb, vc32, vc16,
             sem_w, sem_x, sem_o, sem_a, sem_b, sem_c, sem_d,
             send_sems, recv_sems):
        my = lax.axis_index("i")

        barrier = pltpu.get_barrier_semaphore()
        for delta in DELTAS:
            pl.semaphore_signal(
                barrier, inc=1,
                device_id=((my + delta) % N_DEV,),
                device_id_type=pl.DeviceIdType.MESH,
            )
        pl.semaphore_wait(barrier, 2)

        def row0(di, c):
            return di * HALF + (c % N_DEV) * CHUNK

        def hop_rdmas(s):
            rdmas = []
            for di, delta in enumerate(DELTAS):
                c_send = (my - delta * s) % N_DEV
                if s == 0:
                    src = p_ref.at[pl.ds(row0(di, c_send), CHUNK), :]
                else:
                    src = comm_ref.at[di, s - 1]
                rdmas.append(pltpu.make_async_remote_copy(
                    src_ref=src,
                    dst_ref=comm_ref.at[di, s],
                    send_sem=send_sems.at[di, s],
                    recv_sem=recv_sems.at[di, s],
                    device_id=((my + delta) % N_DEV,),
                    device_id_type=pl.DeviceIdType.MESH,
                ))
            return rdmas

        def ag_rdma(t, di):
            t = jnp.minimum(t, N_DEV - 2)
            delta = DELTAS[di]
            sl = pl.ds(row0(di, (my + delta * (1 - t)) % N_DEV), CHUNK)
            return pltpu.make_async_remote_copy(
                src_ref=g_ref.at[sl, :],
                dst_ref=g_ref.at[sl, :],
                send_sem=send_sems.at[di, N_DEV - 1 + t],
                recv_sem=recv_sems.at[di, N_DEV - 1 + t],
                device_id=((my + delta) % N_DEV,),
                device_id_type=pl.DeviceIdType.MESH,
            )

        def chunk_at(j, slot):
            if slot == 0:
                off = jnp.where(j < N_DEV - 1, -j, 1)
            else:
                off = jnp.where(j < N_DEV - 1, j, -1)
            return (my + off) % N_DEV

        def xcopy(j, slot):
            return pltpu.make_async_copy(
                x_ref.at[pl.ds(row0(slot, chunk_at(j, slot)), CHUNK), :],
                vx.at[slot],
                sem_x.at[slot],
            )

        cw = pltpu.make_async_copy(w_ref, vw, sem_w)
        cw.start()
        xcopy(0, 0).start()
        cw.wait()

        NT = CHUNK // TILE

        def add_chunk(mk_a, mk_b, mk_d16, mk_d32=None):
            def ins(i):
                b = i % 2
                return (
                    pltpu.make_async_copy(mk_a(i * TILE), va.at[b], sem_a.at[b]),
                    pltpu.make_async_copy(mk_b(i * TILE), vb.at[b], sem_b.at[b]),
                )
            prev = [None, None]
            for c in ins(0):
                c.start()
            for i in range(NT):
                b = i % 2
                if i + 1 < NT:
                    for c in ins(i + 1):
                        c.start()
                for c in ins(i):
                    c.wait()
                if prev[b] is not None:
                    for c in prev[b]:
                        c.wait()
                if mk_d32 is not None:
                    vc32[b] = (va[b].astype(jnp.float32)
                               + vb[b].astype(jnp.float32))
                    vc16[b] = vc32[b].astype(jnp.bfloat16)
                else:
                    vc16[b] = va[b] + vb[b]
                outs = [pltpu.make_async_copy(
                    vc16.at[b], mk_d16(i * TILE), sem_c.at[b])]
                if mk_d32 is not None:
                    outs.append(pltpu.make_async_copy(
                        vc32.at[b], mk_d32(i * TILE), sem_d.at[b]))
                for c in outs:
                    c.start()
                prev[b] = outs
            for pv in prev:
                if pv is not None:
                    for c in pv:
                        c.wait()

        def gemm_iter(j, _):
            xcopy(j, 1).start()
            for slot in range(2):
                xcopy(j, slot).wait()
                vo[...] = jnp.dot(
                    vx[slot].astype(jnp.bfloat16), vw[...],
                    preferred_element_type=jnp.float32,
                ).astype(jnp.bfloat16)
                co = pltpu.make_async_copy(
                    vo,
                    p_ref.at[pl.ds(row0(slot, chunk_at(j, slot)), CHUNK), :],
                    sem_o)
                co.start()
                co.wait()
                if slot == 0:
                    @pl.when(j < N_DEV - 1)
                    def _():
                        xcopy(j + 1, 0).start()

            @pl.when(j == 0)
            def _():
                for rdma in hop_rdmas(0):
                    rdma.start()

            @pl.when(j == N_DEV - 2)
            def _():
                rd0 = hop_rdmas(0)
                rd1 = hop_rdmas(1)
                for di, delta in enumerate(DELTAS):
                    rd0[di].wait()
                    c_recv = (my - delta) % N_DEV
                    add_chunk(
                        lambda r, di=di: comm_ref.at[di, 0, pl.ds(r, TILE), :],
                        lambda r, di=di, c=c_recv: p_ref.at[
                            pl.ds(row0(di, c) + r, TILE), :],
                        lambda r, di=di: comm_ref.at[di, 0, pl.ds(r, TILE), :],
                    )
                    rd1[di].start()
            return None

        lax.fori_loop(0, N_DEV, gemm_iter, None)

        for s in range(1, N_DEV - 1):
            rdmas = hop_rdmas(s)
            nxt = hop_rdmas(s + 1) if s < N_DEV - 2 else None
            for di, delta in enumerate(DELTAS):
                rdmas[di].wait()
                c_recv = (my - delta * (s + 1)) % N_DEV
                if s < N_DEV - 2:
                    add_chunk(
                        lambda r, di=di, s=s: comm_ref.at[di, s, pl.ds(r, TILE), :],
                        lambda r, di=di, c=c_recv: p_ref.at[
                            pl.ds(row0(di, c) + r, TILE), :],
                        lambda r, di=di, s=s: comm_ref.at[di, s, pl.ds(r, TILE), :],
                    )
                    nxt[di].start()
                else:
                    add_chunk(
                        lambda r, di=di, s=s: comm_ref.at[di, s, pl.ds(r, TILE), :],
                        lambda r, di=di, c=c_recv: p_ref.at[
                            pl.ds(row0(di, c) + r, TILE), :],
                        lambda r, di=di, c=c_recv: g_ref.at[
                            pl.ds(row0(di, c) + r, TILE), :],
                        lambda r, di=di, c=c_recv: out_ref.at[
                            pl.ds(row0(di, c) + r, TILE), :],
                    )
                    ag_rdma(0, di).start()

        def cast_chunk(c_row0):
            def ins(i):
                b = i % 2
                return pltpu.make_async_copy(
                    g_ref.at[pl.ds(c_row0 + i * TILE, TILE), :],
                    va.at[b], sem_a.at[b])
            prev = [None, None]
            ins(0).start()
            for i in range(NT):
                b = i % 2
                if i + 1 < NT:
                    ins(i + 1).start()
                ins(i).wait()
                if prev[b] is not None:
                    prev[b].wait()
                vc32[b] = va[b].astype(jnp.float32)
                cd = pltpu.make_async_copy(
                    vc32.at[b],
                    out_ref.at[pl.ds(c_row0 + i * TILE, TILE), :],
                    sem_d.at[b])
                cd.start()
                prev[b] = cd
            for pv in prev:
                if pv is not None:
                    pv.wait()

        def ag_iter(t, _):
            for di in range(2):
                ag_rdma(t, di).wait()

                @pl.when(t < N_DEV - 2)
                def _(di=di):
                    ag_rdma(t + 1, di).start()
            for di, delta in enumerate(DELTAS):
                cast_chunk(row0(di, (my - delta * t) % N_DEV))
            return None

        lax.fori_loop(0, N_DEV - 1, ag_iter, None)

    out, _p, _g, _comm = pl.pallas_call(
        body,
        out_shape=[
            jax.ShapeDtypeStruct((M, N), jnp.float32),
            jax.ShapeDtypeStruct((M, N), jnp.bfloat16),
            jax.ShapeDtypeStruct((M, N), jnp.bfloat16),
            jax.ShapeDtypeStruct((2, N_DEV - 1, CHUNK, N), jnp.bfloat16),
        ],
        in_specs=[
            pl.BlockSpec(memory_space=pl.ANY),
            pl.BlockSpec(memory_space=pl.ANY),
        ],
        out_specs=[
            pl.BlockSpec(memory_space=pl.ANY),
            pl.BlockSpec(memory_space=pl.ANY),
            pl.BlockSpec(memory_space=pl.ANY),
            pl.BlockSpec(memory_space=pl.ANY),
        ],
        scratch_shapes=[
            pltpu.VMEM((K_SH, N), jnp.bfloat16),
            pltpu.VMEM((2, CHUNK, K_SH), jnp.float32),
            pltpu.VMEM((CHUNK, N), jnp.bfloat16),
            pltpu.VMEM((2, TILE, N), jnp.bfloat16),
            pltpu.VMEM((2, TILE, N), jnp.bfloat16),
            pltpu.VMEM((2, TILE, N), jnp.float32),
            pltpu.VMEM((2, TILE, N), jnp.bfloat16),
            pltpu.SemaphoreType.DMA,
            pltpu.SemaphoreType.DMA((2,)),
            pltpu.SemaphoreType.DMA,
            pltpu.SemaphoreType.DMA((2,)),
            pltpu.SemaphoreType.DMA((2,)),
            pltpu.SemaphoreType.DMA((2,)),
            pltpu.SemaphoreType.DMA((2,)),
            pltpu.SemaphoreType.DMA((2, 2 * (N_DEV - 1))),
            pltpu.SemaphoreType.DMA((2, 2 * (N_DEV - 1))),
        ],
        compiler_params=pltpu.CompilerParams(
            collective_id=0, vmem_limit_bytes=64 * 1024 * 1024),
    )(x, w16)
    return out
